# baseline (speedup 1.0000x reference)
import functools

import jax
import jax.numpy as jnp
from jax import lax
from jax.experimental import pallas as pl
from jax.experimental.pallas import tpu as pltpu
from jax.experimental.pallas import tpu_sc as plsc

_N = 2048
_E = 32768
_EH = _E // 2
_SD = 32
_H = 64
_LD = 128
_GW = 2 * _H
_OUT = 2
_MAX_ITER = 5
_THR2 = 1e-18

_NC, _NS = 2, 16
_NW = _NC * _NS
_CH = 128

_EB = 2048
_NBH = _EH // _EB
_EB1 = 1024
_NBH1 = _EH // _EB1


def _sc_gather_rows(table, idx):
    t_rows, d = table.shape
    b = idx.shape[0]
    bpw = b // _NW
    slab = min(bpw, 512)
    nslab = bpw // slab
    nch = slab // _CH
    mesh = plsc.VectorSubcoreMesh(core_axis_name="c", subcore_axis_name="s")

    @functools.partial(
        pl.kernel,
        mesh=mesh,
        out_type=jax.ShapeDtypeStruct((b, d), jnp.float32),
        scratch_types=[
            pltpu.VMEM((bpw,), jnp.int32),
            pltpu.VMEM((slab, d), jnp.float32),
            pltpu.SemaphoreType.DMA,
        ],
    )
    def gather_kernel(table_hbm, idx_hbm, out_hbm, idx_v, rows_v, sem):
        wid = lax.axis_index("s") * _NC + lax.axis_index("c")
        base = wid * bpw
        pltpu.sync_copy(idx_hbm.at[pl.ds(base, bpw)], idx_v)
        for s in range(nslab):
            copies = [
                pltpu.async_copy(
                    table_hbm.at[idx_v.at[pl.ds(s * slab + j * _CH, _CH)]],
                    rows_v.at[pl.ds(j * _CH, _CH), :],
                    sem,
                )
                for j in range(nch)
            ]
            for c in copies:
                c.wait()
            pltpu.sync_copy(rows_v, out_hbm.at[pl.ds(base + s * slab, slab)])

    return gather_kernel(table, idx)


def _prep(node_labels, node_states, W1, b1r):
    def body(lab_ref, st_ref, w1_ref, b1_ref, lbl_ref, tgt_ref, ext_ref):
        lab = lab_ref[...]
        src_t = jnp.dot(lab, w1_ref[0:_LD, :],
                        preferred_element_type=jnp.float32)
        tgt_t = jnp.dot(lab, w1_ref[_LD:2 * _LD, :],
                        preferred_element_type=jnp.float32) + b1_ref[...]
        proj0 = jnp.dot(st_ref[...], w1_ref[2 * _LD:, :],
                        preferred_element_type=jnp.float32)
        lbl_ref[...] = jnp.concatenate([src_t, tgt_t], axis=1)
        tgt_ref[...] = tgt_t
        ext_ref[...] = jnp.concatenate([tgt_t, proj0], axis=1)

    return pl.pallas_call(
        body,
        out_shape=(
            jax.ShapeDtypeStruct((_N, _GW), jnp.float32),
            jax.ShapeDtypeStruct((_N, _H), jnp.float32),
            jax.ShapeDtypeStruct((_N, _GW), jnp.float32),
        ),
    )(node_labels, node_states, W1, b1r)


def _src_part(a):
    def body(a_ref, o_ref):
        o_ref[...] = a_ref[:, 0:_H].astype(jnp.bfloat16)

    rows = a.shape[0]
    blk_rows = rows // 2
    return pl.pallas_call(
        body,
        grid=(2,),
        in_specs=[pl.BlockSpec((blk_rows, _GW), lambda i: (i, 0))],
        out_specs=pl.BlockSpec((blk_rows, _H), lambda i: (i, 0)),
        out_shape=jax.ShapeDtypeStruct((rows, _H), jnp.bfloat16),
    )(a)


def _edge_block(g, sp, w2, b2):
    h = jnp.tanh(sp.astype(jnp.float32) + g[:, 0:_H] + g[:, _H:_GW])
    es = jnp.dot(h, w2, preferred_element_type=jnp.float32)
    return (es + b2).astype(jnp.bfloat16)


def _step_partial(agg_half, g, sp, W2, b2r):
    def body(agg_ref, g_ref, sp_ref, w2_ref, b2_ref, acc_out_ref, acc_ref):
        j = pl.program_id(0)
        es = _edge_block(g_ref[...], sp_ref[...], w2_ref[...], b2_ref[...])
        contrib = jnp.dot(agg_ref[...], es, preferred_element_type=jnp.float32)

        @pl.when(j == 0)
        def _init():
            acc_ref[...] = contrib

        @pl.when(j > 0)
        def _accum():
            acc_ref[...] += contrib

        @pl.when(j == _NBH - 1)
        def _finish():
            acc_out_ref[...] = acc_ref[...]

    return pl.pallas_call(
        body,
        grid=(_NBH,),
        in_specs=[
            pl.BlockSpec((_N, _EB), lambda j: (0, j)),
            pl.BlockSpec((_EB, _GW), lambda j: (j, 0)),
            pl.BlockSpec((_EB, _H), lambda j: (j, 0)),
            pl.BlockSpec((_H, _SD), lambda j: (0, 0)),
            pl.BlockSpec((1, _SD), lambda j: (0, 0)),
        ],
        out_specs=pl.BlockSpec((_N, _SD), lambda j: (0, 0)),
        out_shape=jax.ShapeDtypeStruct((_N, _SD), jnp.float32),
        scratch_shapes=[pltpu.VMEM((_N, _SD), jnp.float32)],
        compiler_params=pltpu.CompilerParams(
            dimension_semantics=("arbitrary",)),
    )(agg_half, g, sp, W2, b2r)


def _step_final(agg_half, g, sp, W2, b2r, W1s, tgt_half, states_old, acc_in,
                goff=0):
    def body(agg_ref, g_ref, sp_ref, w2_ref, b2_ref, w1s_ref, th_ref,
             old_ref, accin_ref, ns_ref, ext_ref, done_ref, acc_ref):
        j = pl.program_id(0)
        es = _edge_block(g_ref[...], sp_ref[...], w2_ref[...], b2_ref[...])
        contrib = jnp.dot(agg_ref[...], es, preferred_element_type=jnp.float32)

        @pl.when(j == 0)
        def _init():
            acc_ref[...] = accin_ref[...] + contrib

        @pl.when(j > 0)
        def _accum():
            acc_ref[...] += contrib

        @pl.when(j == _NBH - 1)
        def _finish():
            new_s = acc_ref[...]
            ns_ref[...] = new_s
            new_p = jnp.dot(new_s, w1s_ref[...],
                            preferred_element_type=jnp.float32)
            ext_ref[...] = jnp.concatenate([th_ref[...], new_p], axis=1)
            diff = new_s - old_ref[...]
            dist2 = jnp.sum(diff * diff, axis=1, keepdims=True)
            done_ref[0, 0] = jnp.where(
                jnp.max(dist2) < _THR2, 1, 0).astype(jnp.int32)

    return pl.pallas_call(
        body,
        grid=(_NBH,),
        in_specs=[
            pl.BlockSpec((_N, _EB), lambda j: (0, j)),
            pl.BlockSpec((_EB, _GW), lambda j: (j + goff, 0)),
            pl.BlockSpec((_EB, _H), lambda j: (j + goff, 0)),
            pl.BlockSpec((_H, _SD), lambda j: (0, 0)),
            pl.BlockSpec((1, _SD), lambda j: (0, 0)),
            pl.BlockSpec((_SD, _H), lambda j: (0, 0)),
            pl.BlockSpec((_N, _H), lambda j: (0, 0)),
            pl.BlockSpec((_N, _SD), lambda j: (0, 0)),
            pl.BlockSpec((_N, _SD), lambda j: (0, 0)),
        ],
        out_specs=(
            pl.BlockSpec((_N, _SD), lambda j: (0, 0)),
            pl.BlockSpec((_N, _GW), lambda j: (0, 0)),
            pl.BlockSpec((1, 1), lambda j: (0, 0), memory_space=pltpu.SMEM),
        ),
        out_shape=(
            jax.ShapeDtypeStruct((_N, _SD), jnp.float32),
            jax.ShapeDtypeStruct((_N, _GW), jnp.float32),
            jax.ShapeDtypeStruct((1, 1), jnp.int32),
        ),
        scratch_shapes=[pltpu.VMEM((_N, _SD), jnp.float32)],
        compiler_params=pltpu.CompilerParams(
            dimension_semantics=("arbitrary",)),
    )(agg_half, g, sp, W2, b2r, W1s, tgt_half, states_old, acc_in)


def _step_cast_partial(agg, g, sp, W2, b2r):
    def body(agg_ref, g_ref, sp_ref, w2_ref, b2_ref, ab_ref, acc_out_ref,
             acc_ref):
        j = pl.program_id(0)
        es = _edge_block(g_ref[...], sp_ref[...], w2_ref[...], b2_ref[...])
        a_bf = agg_ref[...].astype(jnp.bfloat16)
        ab_ref[...] = a_bf
        contrib = jnp.dot(a_bf, es, preferred_element_type=jnp.float32)

        @pl.when(j == 0)
        def _init():
            acc_ref[...] = contrib

        @pl.when(j > 0)
        def _accum():
            acc_ref[...] += contrib

        @pl.when(j == _NBH1 - 1)
        def _finish():
            acc_out_ref[...] = acc_ref[...]

    return pl.pallas_call(
        body,
        grid=(_NBH1,),
        in_specs=[
            pl.BlockSpec((_N, _EB1), lambda j: (0, j)),
            pl.BlockSpec((_EB1, _GW), lambda j: (j, 0)),
            pl.BlockSpec((_EB1, _H), lambda j: (j, 0)),
            pl.BlockSpec((_H, _SD), lambda j: (0, 0)),
            pl.BlockSpec((1, _SD), lambda j: (0, 0)),
        ],
        out_specs=(
            pl.BlockSpec((_N, _EB1), lambda j: (0, j)),
            pl.BlockSpec((_N, _SD), lambda j: (0, 0)),
        ),
        out_shape=(
            jax.ShapeDtypeStruct((_N, _EH), jnp.bfloat16),
            jax.ShapeDtypeStruct((_N, _SD), jnp.float32),
        ),
        scratch_shapes=[pltpu.VMEM((_N, _SD), jnp.float32)],
        compiler_params=pltpu.CompilerParams(
            dimension_semantics=("arbitrary",)),
    )(agg, g, sp, W2, b2r)


def _step_cast_final(agg, g, sp, W2, b2r, W1s, tgt_half, states_old, acc_in,
                     goff=0):
    def body(agg_ref, g_ref, sp_ref, w2_ref, b2_ref, w1s_ref, th_ref,
             old_ref, accin_ref, ab_ref, ns_ref, ext_ref, done_ref, acc_ref):
        j = pl.program_id(0)
        es = _edge_block(g_ref[...], sp_ref[...], w2_ref[...], b2_ref[...])
        a_bf = agg_ref[...].astype(jnp.bfloat16)
        ab_ref[...] = a_bf
        contrib = jnp.dot(a_bf, es, preferred_element_type=jnp.float32)

        @pl.when(j == 0)
        def _init():
            acc_ref[...] = accin_ref[...] + contrib

        @pl.when(j > 0)
        def _accum():
            acc_ref[...] += contrib

        @pl.when(j == _NBH1 - 1)
        def _finish():
            new_s = acc_ref[...]
            ns_ref[...] = new_s
            new_p = jnp.dot(new_s, w1s_ref[...],
                            preferred_element_type=jnp.float32)
            ext_ref[...] = jnp.concatenate([th_ref[...], new_p], axis=1)
            diff = new_s - old_ref[...]
            dist2 = jnp.sum(diff * diff, axis=1, keepdims=True)
            done_ref[0, 0] = jnp.where(
                jnp.max(dist2) < _THR2, 1, 0).astype(jnp.int32)

    return pl.pallas_call(
        body,
        grid=(_NBH1,),
        in_specs=[
            pl.BlockSpec((_N, _EB1), lambda j: (0, j + _NBH1)),
            pl.BlockSpec((_EB1, _GW), lambda j: (j + goff, 0)),
            pl.BlockSpec((_EB1, _H), lambda j: (j + goff, 0)),
            pl.BlockSpec((_H, _SD), lambda j: (0, 0)),
            pl.BlockSpec((1, _SD), lambda j: (0, 0)),
            pl.BlockSpec((_SD, _H), lambda j: (0, 0)),
            pl.BlockSpec((_N, _H), lambda j: (0, 0)),
            pl.BlockSpec((_N, _SD), lambda j: (0, 0)),
            pl.BlockSpec((_N, _SD), lambda j: (0, 0)),
        ],
        out_specs=(
            pl.BlockSpec((_N, _EB1), lambda j: (0, j)),
            pl.BlockSpec((_N, _SD), lambda j: (0, 0)),
            pl.BlockSpec((_N, _GW), lambda j: (0, 0)),
            pl.BlockSpec((1, 1), lambda j: (0, 0), memory_space=pltpu.SMEM),
        ),
        out_shape=(
            jax.ShapeDtypeStruct((_N, _EH), jnp.bfloat16),
            jax.ShapeDtypeStruct((_N, _SD), jnp.float32),
            jax.ShapeDtypeStruct((_N, _GW), jnp.float32),
            jax.ShapeDtypeStruct((1, 1), jnp.int32),
        ),
        scratch_shapes=[pltpu.VMEM((_N, _SD), jnp.float32)],
        compiler_params=pltpu.CompilerParams(
            dimension_semantics=("arbitrary",)),
    )(agg, g, sp, W2, b2r, W1s, tgt_half, states_old, acc_in)


def _out_mlp(states, Wo1, bo1r, Wo2, bo2r):
    def body(st_ref, w1_ref, b1_ref, w2_ref, b2_ref, o_ref):
        hid = jnp.tanh(
            jnp.dot(st_ref[...], w1_ref[...], preferred_element_type=jnp.float32)
            + b1_ref[...])
        o_ref[...] = jnp.dot(
            hid, w2_ref[...], preferred_element_type=jnp.float32) + b2_ref[...]

    return pl.pallas_call(
        body,
        out_shape=jax.ShapeDtypeStruct((_N, _OUT), jnp.float32),
    )(states, Wo1, bo1r, Wo2, bo2r)


def kernel(edges, agg_matrix, node_labels, node_states, W1, b1, W2, b2,
           Wo1, bo1, Wo2, bo2):
    src_idx = edges[:, 0].astype(jnp.int32)
    tgt_idx = edges[:, 1].astype(jnp.int32)
    W1s = W1[2 * _LD:, :]
    b2r = b2.reshape(1, _SD)

    lbl_tab, tgt_half, ext0 = _prep(node_labels, node_states, W1,
                                    b1.reshape(1, _H))
    sp = _src_part(_sc_gather_rows(lbl_tab, src_idx))
    g0 = _sc_gather_rows(ext0, tgt_idx)

    agg_a, acc0 = _step_cast_partial(agg_matrix, g0, sp, W2, b2r)
    agg_b, s1, ext1, done0 = _step_cast_final(
        agg_matrix, g0, sp, W2, b2r, W1s, tgt_half, node_states, acc0,
        goff=_NBH1)

    def cond_fun(carry):
        _, _, n_it, done = carry
        return jnp.logical_and(n_it < _MAX_ITER, jnp.logical_not(done))

    def body_fun(carry):
        states, ext, n_it, _ = carry
        g = _sc_gather_rows(ext, tgt_idx)
        acc = _step_partial(agg_a, g, sp, W2, b2r)
        new_s, new_ext, done_i = _step_final(
            agg_b, g, sp, W2, b2r, W1s, tgt_half, states, acc, goff=_NBH)
        return (new_s, new_ext, n_it + 1, done_i[0, 0] != 0)

    states, _, n_it, _ = lax.while_loop(
        cond_fun, body_fun,
        (s1, ext1, jnp.asarray(1, jnp.int32), done0[0, 0] != 0))

    out = _out_mlp(states, Wo1, bo1.reshape(1, _H), Wo2,
                   bo2.reshape(1, _OUT))
    return (out, jnp.asarray(n_it, jnp.int32))


# speedup vs baseline: 1.0561x; 1.0561x over previous
import functools

import jax
import jax.numpy as jnp
from jax import lax
from jax.experimental import pallas as pl
from jax.experimental.pallas import tpu as pltpu
from jax.experimental.pallas import tpu_sc as plsc

_N = 2048
_E = 32768
_SD = 32
_H = 64
_LD = 128
_GW = 2 * _H
_OUT = 2
_MAX_ITER = 5
_THR2 = 1e-18

_NC, _NS = 2, 16
_NW = _NC * _NS
_CH = 128

_EB = 4096
_NB = _E // _EB
_EB1 = 2048
_NB1 = _E // _EB1


def _sc_gather_rows(table, idx):
    t_rows, d = table.shape
    b = idx.shape[0]
    bpw = b // _NW
    slab = min(bpw, 512)
    nslab = bpw // slab
    nch = slab // _CH
    mesh = plsc.VectorSubcoreMesh(core_axis_name="c", subcore_axis_name="s")

    @functools.partial(
        pl.kernel,
        mesh=mesh,
        out_type=jax.ShapeDtypeStruct((b, d), jnp.float32),
        scratch_types=[
            pltpu.VMEM((bpw,), jnp.int32),
            pltpu.VMEM((slab, d), jnp.float32),
            pltpu.SemaphoreType.DMA,
        ],
    )
    def gather_kernel(table_hbm, idx_hbm, out_hbm, idx_v, rows_v, sem):
        wid = lax.axis_index("s") * _NC + lax.axis_index("c")
        base = wid * bpw
        pltpu.sync_copy(idx_hbm.at[pl.ds(base, bpw)], idx_v)
        for s in range(nslab):
            copies = [
                pltpu.async_copy(
                    table_hbm.at[idx_v.at[pl.ds(s * slab + j * _CH, _CH)]],
                    rows_v.at[pl.ds(j * _CH, _CH), :],
                    sem,
                )
                for j in range(nch)
            ]
            for c in copies:
                c.wait()
            pltpu.sync_copy(rows_v, out_hbm.at[pl.ds(base + s * slab, slab)])

    return gather_kernel(table, idx)


def _prep(node_labels, node_states, W1, b1r):
    def body(lab_ref, st_ref, w1_ref, b1_ref, lbl_ref, tgt_ref, ext_ref):
        lab = lab_ref[...]
        src_t = jnp.dot(lab, w1_ref[0:_LD, :],
                        preferred_element_type=jnp.float32)
        tgt_t = jnp.dot(lab, w1_ref[_LD:2 * _LD, :],
                        preferred_element_type=jnp.float32) + b1_ref[...]
        proj0 = jnp.dot(st_ref[...], w1_ref[2 * _LD:, :],
                        preferred_element_type=jnp.float32)
        lbl_ref[...] = jnp.concatenate([src_t, tgt_t], axis=1)
        tgt_ref[...] = tgt_t
        ext_ref[...] = jnp.concatenate([tgt_t, proj0], axis=1)

    return pl.pallas_call(
        body,
        out_shape=(
            jax.ShapeDtypeStruct((_N, _GW), jnp.float32),
            jax.ShapeDtypeStruct((_N, _H), jnp.float32),
            jax.ShapeDtypeStruct((_N, _GW), jnp.float32),
        ),
    )(node_labels, node_states, W1, b1r)


def _src_part(a):
    def body(a_ref, o_ref):
        o_ref[...] = a_ref[:, 0:_H].astype(jnp.bfloat16)

    rows = a.shape[0]
    blk_rows = rows // 2
    return pl.pallas_call(
        body,
        grid=(2,),
        in_specs=[pl.BlockSpec((blk_rows, _GW), lambda i: (i, 0))],
        out_specs=pl.BlockSpec((blk_rows, _H), lambda i: (i, 0)),
        out_shape=jax.ShapeDtypeStruct((rows, _H), jnp.bfloat16),
    )(a)


def _edge_block(g, sp, w2, b2):
    h = jnp.tanh(sp.astype(jnp.float32) + g[:, 0:_H] + g[:, _H:_GW])
    es = jnp.dot(h, w2, preferred_element_type=jnp.float32)
    return (es + b2).astype(jnp.bfloat16)


def _step(agg_bf, g, sp, W2, b2r, W1s, tgt_half, states_old):
    def body(agg_ref, g_ref, sp_ref, w2_ref, b2_ref, w1s_ref, th_ref,
             old_ref, ns_ref, ext_ref, done_ref, acc_ref):
        j = pl.program_id(0)
        es = _edge_block(g_ref[...], sp_ref[...], w2_ref[...], b2_ref[...])
        contrib = jnp.dot(agg_ref[...], es, preferred_element_type=jnp.float32)

        @pl.when(j == 0)
        def _init():
            acc_ref[...] = contrib

        @pl.when(j > 0)
        def _accum():
            acc_ref[...] += contrib

        @pl.when(j == _NB - 1)
        def _finish():
            new_s = acc_ref[...]
            ns_ref[...] = new_s
            new_p = jnp.dot(new_s, w1s_ref[...],
                            preferred_element_type=jnp.float32)
            ext_ref[...] = jnp.concatenate([th_ref[...], new_p], axis=1)
            diff = new_s - old_ref[...]
            dist2 = jnp.sum(diff * diff, axis=1, keepdims=True)
            done_ref[0, 0] = jnp.where(
                jnp.max(dist2) < _THR2, 1, 0).astype(jnp.int32)

    return pl.pallas_call(
        body,
        grid=(_NB,),
        in_specs=[
            pl.BlockSpec((_N, _EB), lambda j: (0, j)),
            pl.BlockSpec((_EB, _GW), lambda j: (j, 0)),
            pl.BlockSpec((_EB, _H), lambda j: (j, 0)),
            pl.BlockSpec((_H, _SD), lambda j: (0, 0)),
            pl.BlockSpec((1, _SD), lambda j: (0, 0)),
            pl.BlockSpec((_SD, _H), lambda j: (0, 0)),
            pl.BlockSpec((_N, _H), lambda j: (0, 0)),
            pl.BlockSpec((_N, _SD), lambda j: (0, 0)),
        ],
        out_specs=(
            pl.BlockSpec((_N, _SD), lambda j: (0, 0)),
            pl.BlockSpec((_N, _GW), lambda j: (0, 0)),
            pl.BlockSpec((1, 1), lambda j: (0, 0), memory_space=pltpu.SMEM),
        ),
        out_shape=(
            jax.ShapeDtypeStruct((_N, _SD), jnp.float32),
            jax.ShapeDtypeStruct((_N, _GW), jnp.float32),
            jax.ShapeDtypeStruct((1, 1), jnp.int32),
        ),
        scratch_shapes=[pltpu.VMEM((_N, _SD), jnp.float32)],
        compiler_params=pltpu.CompilerParams(
            dimension_semantics=("arbitrary",)),
    )(agg_bf, g, sp, W2, b2r, W1s, tgt_half, states_old)


def _step_cast(agg, g, sp, W2, b2r, W1s, tgt_half, states_old):
    def body(agg_ref, g_ref, sp_ref, w2_ref, b2_ref, w1s_ref, th_ref,
             old_ref, ab_ref, ns_ref, ext_ref, done_ref, acc_ref):
        j = pl.program_id(0)
        es = _edge_block(g_ref[...], sp_ref[...], w2_ref[...], b2_ref[...])
        a_bf = agg_ref[...].astype(jnp.bfloat16)
        ab_ref[...] = a_bf
        contrib = jnp.dot(a_bf, es, preferred_element_type=jnp.float32)

        @pl.when(j == 0)
        def _init():
            acc_ref[...] = contrib

        @pl.when(j > 0)
        def _accum():
            acc_ref[...] += contrib

        @pl.when(j == _NB1 - 1)
        def _finish():
            new_s = acc_ref[...]
            ns_ref[...] = new_s
            new_p = jnp.dot(new_s, w1s_ref[...],
                            preferred_element_type=jnp.float32)
            ext_ref[...] = jnp.concatenate([th_ref[...], new_p], axis=1)
            diff = new_s - old_ref[...]
            dist2 = jnp.sum(diff * diff, axis=1, keepdims=True)
            done_ref[0, 0] = jnp.where(
                jnp.max(dist2) < _THR2, 1, 0).astype(jnp.int32)

    return pl.pallas_call(
        body,
        grid=(_NB1,),
        in_specs=[
            pl.BlockSpec((_N, _EB1), lambda j: (0, j)),
            pl.BlockSpec((_EB1, _GW), lambda j: (j, 0)),
            pl.BlockSpec((_EB1, _H), lambda j: (j, 0)),
            pl.BlockSpec((_H, _SD), lambda j: (0, 0)),
            pl.BlockSpec((1, _SD), lambda j: (0, 0)),
            pl.BlockSpec((_SD, _H), lambda j: (0, 0)),
            pl.BlockSpec((_N, _H), lambda j: (0, 0)),
            pl.BlockSpec((_N, _SD), lambda j: (0, 0)),
        ],
        out_specs=(
            pl.BlockSpec((_N, _EB1), lambda j: (0, j)),
            pl.BlockSpec((_N, _SD), lambda j: (0, 0)),
            pl.BlockSpec((_N, _GW), lambda j: (0, 0)),
            pl.BlockSpec((1, 1), lambda j: (0, 0), memory_space=pltpu.SMEM),
        ),
        out_shape=(
            jax.ShapeDtypeStruct((_N, _E), jnp.bfloat16),
            jax.ShapeDtypeStruct((_N, _SD), jnp.float32),
            jax.ShapeDtypeStruct((_N, _GW), jnp.float32),
            jax.ShapeDtypeStruct((1, 1), jnp.int32),
        ),
        scratch_shapes=[pltpu.VMEM((_N, _SD), jnp.float32)],
        compiler_params=pltpu.CompilerParams(
            dimension_semantics=("arbitrary",)),
    )(agg, g, sp, W2, b2r, W1s, tgt_half, states_old)


def _out_mlp(states, Wo1, bo1r, Wo2, bo2r):
    def body(st_ref, w1_ref, b1_ref, w2_ref, b2_ref, o_ref):
        hid = jnp.tanh(
            jnp.dot(st_ref[...], w1_ref[...], preferred_element_type=jnp.float32)
            + b1_ref[...])
        o_ref[...] = jnp.dot(
            hid, w2_ref[...], preferred_element_type=jnp.float32) + b2_ref[...]

    return pl.pallas_call(
        body,
        out_shape=jax.ShapeDtypeStruct((_N, _OUT), jnp.float32),
    )(states, Wo1, bo1r, Wo2, bo2r)


def kernel(edges, agg_matrix, node_labels, node_states, W1, b1, W2, b2,
           Wo1, bo1, Wo2, bo2):
    src_idx = edges[:, 0].astype(jnp.int32)
    tgt_idx = edges[:, 1].astype(jnp.int32)
    W1s = W1[2 * _LD:, :]
    b2r = b2.reshape(1, _SD)

    lbl_tab, tgt_half, ext0 = _prep(node_labels, node_states, W1,
                                    b1.reshape(1, _H))
    sp = _src_part(_sc_gather_rows(lbl_tab, src_idx))
    g0 = _sc_gather_rows(ext0, tgt_idx)

    agg_bf, s1, ext1, done0 = _step_cast(
        agg_matrix, g0, sp, W2, b2r, W1s, tgt_half, node_states)

    def cond_fun(carry):
        _, _, n_it, done = carry
        return jnp.logical_and(n_it < _MAX_ITER, jnp.logical_not(done))

    def body_fun(carry):
        states, ext, n_it, _ = carry
        g = _sc_gather_rows(ext, tgt_idx)
        new_s, new_ext, done_i = _step(
            agg_bf, g, sp, W2, b2r, W1s, tgt_half, states)
        return (new_s, new_ext, n_it + 1, done_i[0, 0] != 0)

    states, _, n_it, _ = lax.while_loop(
        cond_fun, body_fun,
        (s1, ext1, jnp.asarray(1, jnp.int32), done0[0, 0] != 0))

    out = _out_mlp(states, Wo1, bo1.reshape(1, _H), Wo2,
                   bo2.reshape(1, _OUT))
    return (out, jnp.asarray(n_it, jnp.int32))
